# baseline (speedup 1.0000x reference)
import jax
import jax.numpy as jnp
from jax.experimental import pallas as pl
from jax.experimental.pallas import tpu as pltpu

_GAMMA1 = 1.0
_NB = 16
_SB = 8


def _score_kernel(x_ref, d_ref, c_ref, label_ref, bs_ref):
    n = pl.program_id(0)
    x = x_ref[...]
    ss, css = [], []
    for j in range(_NB):
        dn = d_ref[j]
        cs = jnp.dot(x, dn, preferred_element_type=jnp.float32,
                     precision=jax.lax.Precision.DEFAULT)
        g = jnp.dot(dn.T, dn, preferred_element_type=jnp.float32,
                    precision=jax.lax.Precision.DEFAULT)
        t = jnp.dot(cs, g, preferred_element_type=jnp.float32,
                    precision=jax.lax.Precision.DEFAULT)
        ss.append(jnp.sum(cs * (t - 2.0 * cs), axis=1, keepdims=True))
        css.append(cs)

    s, c = ss[0], css[0]
    lbl = jnp.zeros_like(s, dtype=jnp.int32) + _NB * n
    for j in range(1, _NB):
        better = ss[j] < s
        s = jnp.where(better, ss[j], s)
        c = jnp.where(better, css[j], c)
        lbl = jnp.where(better, _NB * n + j, lbl)

    @pl.when(n == 0)
    def _():
        bs_ref[...] = s
        c_ref[...] = c
        label_ref[...] = lbl

    @pl.when(n > 0)
    def _():
        better = s < bs_ref[...]
        bs_ref[...] = jnp.where(better, s, bs_ref[...])
        c_ref[...] = jnp.where(better, c, c_ref[...])
        label_ref[...] = jnp.where(better, lbl, label_ref[...])


def _recon_kernel(lbl_ref, x_ref, c_ref, *refs):
    d_refs = refs[:_SB]
    xrec_ref, loss_ref, acc_ref = refs[_SB:]
    i = pl.program_id(0)
    num_i = pl.num_programs(0)
    x = x_ref[...]
    ys = []
    for j in range(_SB):
        cj = c_ref[j:j + 1, :]
        dj = d_refs[j][0]
        y = jax.lax.dot_general(
            cj, dj, (((1,), (1,)), ((), ())),
            preferred_element_type=jnp.float32,
            precision=jax.lax.Precision.DEFAULT)
        ys.append(y)
    y_all = jnp.concatenate(ys, axis=0)
    xrec_ref[...] = y_all
    r = y_all - x
    r2 = r * r
    part = r2[:, :128] + r2[:, 128:]

    @pl.when(i == 0)
    def _():
        acc_ref[...] = part

    @pl.when(i > 0)
    def _():
        acc_ref[...] = acc_ref[...] + part

    @pl.when(i == num_i - 1)
    def _():
        tot = jnp.sum(acc_ref[...])
        loss_ref[...] = (tot * (_GAMMA1 / (num_i * _SB * x.shape[1]))).reshape(1, 1)


def kernel(x, D):
    B, dim = x.shape
    N, _, F = D.shape
    c_best, label = pl.pallas_call(
        _score_kernel,
        grid=(N // _NB,),
        in_specs=[
            pl.BlockSpec((B, dim), lambda n: (0, 0)),
            pl.BlockSpec((_NB, dim, F), lambda n: (n, 0, 0)),
        ],
        out_specs=[
            pl.BlockSpec((B, F), lambda n: (0, 0)),
            pl.BlockSpec((B, 1), lambda n: (0, 0)),
        ],
        out_shape=[
            jax.ShapeDtypeStruct((B, F), jnp.float32),
            jax.ShapeDtypeStruct((B, 1), jnp.int32),
        ],
        scratch_shapes=[pltpu.VMEM((B, 1), jnp.float32)],
    )(x, D)

    lbl_flat = label[:, 0]
    d_specs = [
        pl.BlockSpec((1, dim, F), (lambda i, lbl, j=j: (lbl[_SB * i + j], 0, 0)))
        for j in range(_SB)
    ]
    x_rec, loss = pl.pallas_call(
        _recon_kernel,
        grid_spec=pltpu.PrefetchScalarGridSpec(
            num_scalar_prefetch=1,
            grid=(B // _SB,),
            in_specs=[
                pl.BlockSpec((_SB, dim), lambda i, lbl: (i, 0)),
                pl.BlockSpec((_SB, F), lambda i, lbl: (i, 0)),
            ] + d_specs,
            out_specs=[
                pl.BlockSpec((_SB, dim), lambda i, lbl: (i, 0)),
                pl.BlockSpec((1, 1), lambda i, lbl: (0, 0)),
            ],
            scratch_shapes=[pltpu.VMEM((_SB, 128), jnp.float32)],
        ),
        out_shape=[
            jax.ShapeDtypeStruct((B, dim), jnp.float32),
            jax.ShapeDtypeStruct((1, 1), jnp.float32),
        ],
    )(lbl_flat, x, c_best, *([D] * _SB))
    return (x_rec, loss[0, 0], lbl_flat)


# speedup vs baseline: 1.7457x; 1.7457x over previous
import jax
import jax.numpy as jnp
from jax.experimental import pallas as pl
from jax.experimental.pallas import tpu as pltpu

_GAMMA1 = 1.0
_NB = 32


def _kf_kernel(x_ref, d_ref, xrec_ref, loss_ref, label_ref, bd2_ref):
    n = pl.program_id(0)
    num_n = pl.num_programs(0)
    x = x_ref[...]
    d2s, xhs = [], []
    for j in range(_NB):
        dn = d_ref[j]
        cs = jnp.dot(x, dn, preferred_element_type=jnp.float32,
                     precision=jax.lax.Precision.DEFAULT)
        xh = jnp.dot(cs, dn.T, preferred_element_type=jnp.float32,
                     precision=jax.lax.Precision.DEFAULT)
        diff = xh - x
        d2s.append(jnp.sum(diff * diff, axis=1, keepdims=True))
        xhs.append(xh)

    d2, xh = d2s[0], xhs[0]
    lbl = jnp.zeros_like(d2, dtype=jnp.int32) + _NB * n
    for j in range(1, _NB):
        better = d2s[j] < d2
        d2 = jnp.where(better, d2s[j], d2)
        xh = jnp.where(better, xhs[j], xh)
        lbl = jnp.where(better, _NB * n + j, lbl)

    @pl.when(n == 0)
    def _():
        bd2_ref[...] = d2
        xrec_ref[...] = xh
        label_ref[...] = lbl

    @pl.when(n > 0)
    def _():
        better = d2 < bd2_ref[...]
        bd2_ref[...] = jnp.where(better, d2, bd2_ref[...])
        xrec_ref[...] = jnp.where(better, xh, xrec_ref[...])
        label_ref[...] = jnp.where(better, lbl, label_ref[...])

    @pl.when(n == num_n - 1)
    def _():
        r = xrec_ref[...] - x
        loss_ref[...] = (jnp.mean(r * r) * _GAMMA1).reshape(1, 1)


def kernel(x, D):
    B, dim = x.shape
    N, _, F = D.shape
    x_rec, loss, label = pl.pallas_call(
        _kf_kernel,
        grid=(N // _NB,),
        in_specs=[
            pl.BlockSpec((B, dim), lambda n: (0, 0)),
            pl.BlockSpec((_NB, dim, F), lambda n: (n, 0, 0)),
        ],
        out_specs=[
            pl.BlockSpec((B, dim), lambda n: (0, 0)),
            pl.BlockSpec((1, 1), lambda n: (0, 0)),
            pl.BlockSpec((B, 1), lambda n: (0, 0)),
        ],
        out_shape=[
            jax.ShapeDtypeStruct((B, dim), jnp.float32),
            jax.ShapeDtypeStruct((1, 1), jnp.float32),
            jax.ShapeDtypeStruct((B, 1), jnp.int32),
        ],
        scratch_shapes=[pltpu.VMEM((B, 1), jnp.float32)],
    )(x, D)
    return (x_rec, loss[0, 0], label[:, 0])


# speedup vs baseline: 1.7699x; 1.0138x over previous
import jax
import jax.numpy as jnp
from jax.experimental import pallas as pl
from jax.experimental.pallas import tpu as pltpu

_GAMMA1 = 1.0
_NB = 32


def _kf_kernel(x_ref, d_ref, xrec_ref, loss_ref, label_ref, bd2_ref):
    n = pl.program_id(0)
    num_n = pl.num_programs(0)
    x = x_ref[...]
    d2s, xhs = [], []
    for j in range(_NB):
        dn = d_ref[j]
        cs = jnp.dot(x, dn, preferred_element_type=jnp.float32,
                     precision=jax.lax.Precision.DEFAULT)
        xh = jnp.dot(cs, dn.T, preferred_element_type=jnp.float32,
                     precision=jax.lax.Precision.DEFAULT)
        diff = xh - x
        d2s.append(jnp.sum(diff * diff, axis=1, keepdims=True))
        xhs.append(xh.astype(jnp.bfloat16))

    d2, xh = d2s[0], xhs[0]
    lbl = jnp.zeros_like(d2, dtype=jnp.int32) + _NB * n
    for j in range(1, _NB):
        better = d2s[j] < d2
        d2 = jnp.where(better, d2s[j], d2)
        xh = jnp.where(better, xhs[j], xh)
        lbl = jnp.where(better, _NB * n + j, lbl)

    @pl.when(n == 0)
    def _():
        bd2_ref[...] = d2
        xrec_ref[...] = xh
        label_ref[...] = lbl

    @pl.when(n > 0)
    def _():
        better = d2 < bd2_ref[...]
        bd2_ref[...] = jnp.where(better, d2, bd2_ref[...])
        xrec_ref[...] = jnp.where(better, xh, xrec_ref[...])
        label_ref[...] = jnp.where(better, lbl, label_ref[...])

    @pl.when(n == num_n - 1)
    def _():
        b, dim = x.shape
        tot = jnp.sum(bd2_ref[...])
        loss_ref[...] = (tot * (_GAMMA1 / (b * dim))).reshape(1, 1)


def kernel(x, D):
    B, dim = x.shape
    N, _, F = D.shape
    x_rec, loss, label = pl.pallas_call(
        _kf_kernel,
        grid=(N // _NB,),
        in_specs=[
            pl.BlockSpec((B, dim), lambda n: (0, 0)),
            pl.BlockSpec((_NB, dim, F), lambda n: (n, 0, 0)),
        ],
        out_specs=[
            pl.BlockSpec((B, dim), lambda n: (0, 0)),
            pl.BlockSpec((1, 1), lambda n: (0, 0)),
            pl.BlockSpec((B, 1), lambda n: (0, 0)),
        ],
        out_shape=[
            jax.ShapeDtypeStruct((B, dim), jnp.bfloat16),
            jax.ShapeDtypeStruct((1, 1), jnp.float32),
            jax.ShapeDtypeStruct((B, 1), jnp.int32),
        ],
        scratch_shapes=[pltpu.VMEM((B, 1), jnp.float32)],
    )(x, D)
    return (x_rec.astype(jnp.float32), loss[0, 0], label[:, 0])


# speedup vs baseline: 1.7709x; 1.0006x over previous
import jax
import jax.numpy as jnp
from jax.experimental import pallas as pl
from jax.experimental.pallas import tpu as pltpu

_GAMMA1 = 1.0
_NB = 32


def _kf_kernel(x_ref, d_ref, xrec_ref, loss_ref, label_ref, bd2_ref):
    n = pl.program_id(0)
    num_n = pl.num_programs(0)
    x = x_ref[...]
    d2 = xh = lbl = None
    for j in range(_NB):
        dn = d_ref[j]
        cs = jnp.dot(x, dn, preferred_element_type=jnp.float32,
                     precision=jax.lax.Precision.DEFAULT)
        xh_j = jnp.dot(cs, dn.T, preferred_element_type=jnp.float32,
                       precision=jax.lax.Precision.DEFAULT)
        diff = xh_j - x
        d2_j = jnp.sum(diff * diff, axis=1, keepdims=True)
        xh_j = xh_j.astype(jnp.bfloat16)
        if j == 0:
            d2, xh = d2_j, xh_j
            lbl = jnp.zeros_like(d2_j, dtype=jnp.int32) + _NB * n
        else:
            better = d2_j < d2
            d2 = jnp.where(better, d2_j, d2)
            xh = jnp.where(better, xh_j, xh)
            lbl = jnp.where(better, _NB * n + j, lbl)

    @pl.when(n == 0)
    def _():
        bd2_ref[...] = d2
        xrec_ref[...] = xh
        label_ref[...] = lbl

    @pl.when(n > 0)
    def _():
        better = d2 < bd2_ref[...]
        bd2_ref[...] = jnp.where(better, d2, bd2_ref[...])
        xrec_ref[...] = jnp.where(better, xh, xrec_ref[...])
        label_ref[...] = jnp.where(better, lbl, label_ref[...])

    @pl.when(n == num_n - 1)
    def _():
        b, dim = x.shape
        tot = jnp.sum(bd2_ref[...])
        loss_ref[...] = (tot * (_GAMMA1 / (b * dim))).reshape(1, 1)


def kernel(x, D):
    B, dim = x.shape
    N, _, F = D.shape
    x_rec, loss, label = pl.pallas_call(
        _kf_kernel,
        grid=(N // _NB,),
        in_specs=[
            pl.BlockSpec((B, dim), lambda n: (0, 0)),
            pl.BlockSpec((_NB, dim, F), lambda n: (n, 0, 0)),
        ],
        out_specs=[
            pl.BlockSpec((B, dim), lambda n: (0, 0)),
            pl.BlockSpec((1, 1), lambda n: (0, 0)),
            pl.BlockSpec((B, 1), lambda n: (0, 0)),
        ],
        out_shape=[
            jax.ShapeDtypeStruct((B, dim), jnp.bfloat16),
            jax.ShapeDtypeStruct((1, 1), jnp.float32),
            jax.ShapeDtypeStruct((B, 1), jnp.int32),
        ],
        scratch_shapes=[pltpu.VMEM((B, 1), jnp.float32)],
    )(x, D)
    return (x_rec.astype(jnp.float32), loss[0, 0], label[:, 0])
